# baseline (speedup 1.0000x reference)
import functools

import jax
import jax.numpy as jnp
from jax import lax
from jax.experimental import pallas as pl
from jax.experimental.pallas import tpu as pltpu
from jax.experimental.pallas import tpu_sc as plsc

_C = 10000
_B = 128
_NODE_IN = 256
_GLOB_IN = 32
_WIDTH = 512

_LANES = 16
_CBLK = _NODE_IN // _LANES
_NG = _C // _LANES
_NW = 32
_GBIG = -(-_NG // _NW)
_GSML = _NG // _NW
_NWBIG = _NG - _GSML * _NW
_QG = 5
_ROWS_MAX = _GBIG * _LANES
_BPAD = 1 << 20
_BROWS = -(-_C // _B)


def _sc_segment_sums(node_feats, batch):
    mesh = plsc.VectorSubcoreMesh(core_axis_name="c", subcore_axis_name="s")

    @functools.partial(
        pl.kernel,
        out_type=jax.ShapeDtypeStruct((_NW, _B, _NODE_IN), jnp.float32),
        mesh=mesh,
        scratch_types=[
            pltpu.VMEM((_ROWS_MAX, _NODE_IN), jnp.float32),
            pltpu.VMEM((_ROWS_MAX,), jnp.int32),
            pltpu.VMEM((_B, _NODE_IN), jnp.float32),
            pltpu.SemaphoreType.DMA,
            pltpu.SemaphoreType.DMA,
            pltpu.SemaphoreType.DMA,
            pltpu.SemaphoreType.DMA,
        ],
    )
    def k(nodes, bat, psum, rows_v, idx_v, acc_v, sem1, sem2, sem3, sem4):
        cid = lax.axis_index("c")
        sid = lax.axis_index("s")
        w = sid * 2 + cid
        gs = jnp.where(w < _NWBIG, _GBIG * w,
                       _GBIG * _NWBIG + _GSML * (w - _NWBIG))
        big = w < _NWBIG
        rs = gs * _LANES

        sems = [sem1, sem2, sem3, sem4]
        qrows = _QG * _LANES

        def q_copy(q, n):
            return (nodes.at[pl.ds(rs + q * qrows, n)],
                    rows_v.at[pl.ds(q * qrows, n)], sems[q])

        for q in range(3):
            pltpu.async_copy(*q_copy(q, qrows))

        @pl.when(big)
        def _():
            pltpu.async_copy(*q_copy(3, qrows))
            pltpu.sync_copy(bat.at[pl.ds(rs, _GBIG * _LANES)],
                            idx_v.at[pl.ds(0, _GBIG * _LANES)])

        @pl.when(jnp.logical_not(big))
        def _():
            pltpu.async_copy(*q_copy(3, qrows - _LANES))
            pltpu.sync_copy(bat.at[pl.ds(rs, _GSML * _LANES)],
                            idx_v.at[pl.ds(0, _GSML * _LANES)])

        zero = jnp.zeros((_LANES,), jnp.float32)

        @plsc.parallel_loop(0, _B)
        def _(r):
            for jj in range(_CBLK):
                acc_v[r, pl.ds(jj * _LANES, _LANES)] = zero

        def load_row(r):
            return [rows_v[r, pl.ds(jj * _LANES, _LANES)]
                    for jj in range(_CBLK)]

        def make_group_body(goff):
            def group_body(g):
                gg = g + goff
                rbase = gg * _LANES
                idvec = idx_v[pl.ds(rbase, _LANES)]
                bs = [idvec[i] for i in range(_LANES)]
                uni = bs[0] == bs[_LANES - 1]
                for i in range(1, _LANES - 1):
                    uni = uni & (bs[0] == bs[i])

                @pl.when(uni)
                def _():
                    ss = load_row(rbase)
                    for i in range(1, _LANES):
                        xs = load_row(rbase + i)
                        ss = [a + b for a, b in zip(ss, xs)]
                    for jj in range(_CBLK):
                        plsc.addupdate(
                            acc_v.at[bs[0], pl.ds(jj * _LANES, _LANES)],
                            ss[jj])

                @pl.when(jnp.logical_not(uni))
                def _():
                    xs = load_row(rbase)
                    for i in range(_LANES):
                        nxt = (load_row(rbase + i + 1)
                               if i < _LANES - 1 else None)
                        for jj in range(_CBLK):
                            plsc.addupdate(
                                acc_v.at[bs[i], pl.ds(jj * _LANES, _LANES)],
                                xs[jj])
                        xs = nxt
            return group_body

        for q in range(3):
            pltpu.make_async_copy(*q_copy(q, qrows)).wait()
            plsc.parallel_loop(0, _QG)(make_group_body(q * _QG))

        @pl.when(big)
        def _():
            pltpu.make_async_copy(*q_copy(3, qrows)).wait()
            plsc.parallel_loop(0, _QG)(make_group_body(3 * _QG))

        @pl.when(jnp.logical_not(big))
        def _():
            pltpu.make_async_copy(*q_copy(3, qrows - _LANES)).wait()
            plsc.parallel_loop(0, _QG - 1)(make_group_body(3 * _QG))

        pltpu.sync_copy(acc_v, psum.at[w])

    return k(node_feats, batch)


def _tc_mlp_body(ps, bp, g, w0, bb0, w1, bb1, w2, bb2, out):
    sums = jnp.sum(ps[...], axis=0)
    ids = bp[...][:, :, None]
    seg = lax.broadcasted_iota(jnp.int32, (_BROWS, _B, _B), 2)
    cnt = jnp.sum((ids == seg).astype(jnp.float32), axis=(0, 1))
    cnt = cnt.reshape(_B, 1)
    mean = sums / jnp.maximum(cnt, 1.0)
    x = jnp.concatenate([mean, g[...]], axis=1)
    h = jnp.dot(x, w0[...], preferred_element_type=jnp.float32)
    h = jnp.maximum(h + bb0[...], 0.0)
    h = jnp.dot(h, w1[...], preferred_element_type=jnp.float32)
    h = jnp.maximum(h + bb1[...], 0.0)
    h = jnp.dot(h, w2[...], preferred_element_type=jnp.float32)
    out[...] = h + bb2[...]


def _tc_mlp(psum, bpad, glob, w0, b0, w1, b1, w2, b2):
    return pl.pallas_call(
        _tc_mlp_body,
        out_shape=jax.ShapeDtypeStruct((_B, _WIDTH), jnp.float32),
    )(psum, bpad, glob, w0, b0, w1, b1, w2, b2)


def kernel(node_feats, edge_index, edge_feats, glob_feats, batch,
           W0, b0, W1, b1, W2, b2):
    del edge_index, edge_feats
    psum = _sc_segment_sums(node_feats, batch)
    bpad = jnp.concatenate(
        [batch, jnp.full((_BROWS * _B - _C,), _BPAD, jnp.int32)]
    ).reshape(_BROWS, _B)
    return _tc_mlp(psum, bpad, glob_feats, W0, b0, W1, b1, W2, b2)


# speedup vs baseline: 1.0913x; 1.0913x over previous
import functools

import jax
import jax.numpy as jnp
from jax import lax
from jax.experimental import pallas as pl
from jax.experimental.pallas import tpu as pltpu
from jax.experimental.pallas import tpu_sc as plsc

_C = 10000
_B = 128
_NODE_IN = 256
_GLOB_IN = 32
_WIDTH = 512

_LANES = 16
_CBLK = _NODE_IN // _LANES
_NG = _C // _LANES
_NW = 32
_GBIG = -(-_NG // _NW)
_GSML = _NG // _NW
_NWBIG = _NG - _GSML * _NW
_QG = 5
_ROWS_MAX = _GBIG * _LANES
_BPAD = 1 << 20
_BROWS = -(-_C // _B)


def _sc_segment_sums(node_feats, batch):
    mesh = plsc.VectorSubcoreMesh(core_axis_name="c", subcore_axis_name="s")

    @functools.partial(
        pl.kernel,
        out_type=jax.ShapeDtypeStruct((_NW, _B, _NODE_IN), jnp.float32),
        mesh=mesh,
        scratch_types=[
            pltpu.VMEM((_ROWS_MAX, _NODE_IN), jnp.float32),
            pltpu.VMEM((_ROWS_MAX,), jnp.int32),
            pltpu.VMEM((_B, _NODE_IN), jnp.float32),
            pltpu.SemaphoreType.DMA,
            pltpu.SemaphoreType.DMA,
            pltpu.SemaphoreType.DMA,
            pltpu.SemaphoreType.DMA,
        ],
    )
    def k(nodes, bat, psum, rows_v, idx_v, acc_v, sem1, sem2, sem3, sem4):
        cid = lax.axis_index("c")
        sid = lax.axis_index("s")
        w = sid * 2 + cid
        gs = jnp.where(w < _NWBIG, _GBIG * w,
                       _GBIG * _NWBIG + _GSML * (w - _NWBIG))
        big = w < _NWBIG
        rs = gs * _LANES

        sems = [sem1, sem2, sem3, sem4]
        qrows = _QG * _LANES

        def q_copy(q, n):
            return (nodes.at[pl.ds(rs + q * qrows, n)],
                    rows_v.at[pl.ds(q * qrows, n)], sems[q])

        for q in range(3):
            pltpu.async_copy(*q_copy(q, qrows))

        @pl.when(big)
        def _():
            pltpu.async_copy(*q_copy(3, qrows))
            pltpu.sync_copy(bat.at[pl.ds(rs, _GBIG * _LANES)],
                            idx_v.at[pl.ds(0, _GBIG * _LANES)])

        @pl.when(jnp.logical_not(big))
        def _():
            pltpu.async_copy(*q_copy(3, qrows - _LANES))
            pltpu.sync_copy(bat.at[pl.ds(rs, _GSML * _LANES)],
                            idx_v.at[pl.ds(0, _GSML * _LANES)])

        zero = jnp.zeros((_LANES,), jnp.float32)

        @plsc.parallel_loop(0, _B)
        def _(r):
            for jj in range(_CBLK):
                acc_v[r, pl.ds(jj * _LANES, _LANES)] = zero

        def load_row(r):
            return [rows_v[r, pl.ds(jj * _LANES, _LANES)]
                    for jj in range(_CBLK)]

        def make_group_body(goff):
            def group_body(g):
                gg = g + goff
                rbase = gg * _LANES
                idvec = idx_v[pl.ds(rbase, _LANES)]
                xs = load_row(rbase)
                for i in range(_LANES):
                    b = idvec[i]
                    nxt = load_row(rbase + i + 1) if i < _LANES - 1 else None
                    for jj in range(_CBLK):
                        plsc.addupdate(
                            acc_v.at[b, pl.ds(jj * _LANES, _LANES)], xs[jj])
                    xs = nxt
            return group_body

        for q in range(3):
            pltpu.make_async_copy(*q_copy(q, qrows)).wait()
            plsc.parallel_loop(0, _QG)(make_group_body(q * _QG))

        @pl.when(big)
        def _():
            pltpu.make_async_copy(*q_copy(3, qrows)).wait()
            plsc.parallel_loop(0, _QG)(make_group_body(3 * _QG))

        @pl.when(jnp.logical_not(big))
        def _():
            pltpu.make_async_copy(*q_copy(3, qrows - _LANES)).wait()
            plsc.parallel_loop(0, _QG - 1)(make_group_body(3 * _QG))

        pltpu.sync_copy(acc_v, psum.at[w])

    return k(node_feats, batch)


def _tc_mlp_body(ps, bp, g, w0, bb0, w1, bb1, w2, bb2, out):
    sums = jnp.sum(ps[...], axis=0)
    ids = bp[...][:, :, None]
    seg = lax.broadcasted_iota(jnp.int32, (_BROWS, _B, _B), 2)
    cnt = jnp.sum((ids == seg).astype(jnp.float32), axis=(0, 1))
    cnt = cnt.reshape(_B, 1)
    mean = sums / jnp.maximum(cnt, 1.0)
    x = jnp.concatenate([mean, g[...]], axis=1)
    h = jnp.dot(x, w0[...], preferred_element_type=jnp.float32)
    h = jnp.maximum(h + bb0[...], 0.0)
    h = jnp.dot(h, w1[...], preferred_element_type=jnp.float32)
    h = jnp.maximum(h + bb1[...], 0.0)
    h = jnp.dot(h, w2[...], preferred_element_type=jnp.float32)
    out[...] = h + bb2[...]


def _tc_mlp(psum, bpad, glob, w0, b0, w1, b1, w2, b2):
    return pl.pallas_call(
        _tc_mlp_body,
        out_shape=jax.ShapeDtypeStruct((_B, _WIDTH), jnp.float32),
    )(psum, bpad, glob, w0, b0, w1, b1, w2, b2)


def kernel(node_feats, edge_index, edge_feats, glob_feats, batch,
           W0, b0, W1, b1, W2, b2):
    del edge_index, edge_feats
    psum = _sc_segment_sums(node_feats, batch)
    bpad = jnp.concatenate(
        [batch, jnp.full((_BROWS * _B - _C,), _BPAD, jnp.int32)]
    ).reshape(_BROWS, _B)
    return _tc_mlp(psum, bpad, glob_feats, W0, b0, W1, b1, W2, b2)


# speedup vs baseline: 1.2159x; 1.1141x over previous
import functools

import jax
import jax.numpy as jnp
from jax import lax
from jax.experimental import pallas as pl
from jax.experimental.pallas import tpu as pltpu
from jax.experimental.pallas import tpu_sc as plsc

_C = 10000
_B = 128
_NODE_IN = 256
_GLOB_IN = 32
_WIDTH = 512

_LANES = 16
_CBLK = _NODE_IN // _LANES
_NG = _C // _LANES
_NW = 32
_GBIG = -(-_NG // _NW)
_GSML = _NG // _NW
_NWBIG = _NG - _GSML * _NW
_H1 = 10
_ROWS_MAX = _GBIG * _LANES
_BPAD = 1 << 20
_BROWS = -(-_C // _B)


def _sc_segment_sums(node_feats, batch):
    mesh = plsc.VectorSubcoreMesh(core_axis_name="c", subcore_axis_name="s")

    @functools.partial(
        pl.kernel,
        out_type=jax.ShapeDtypeStruct((_NW, _B, _NODE_IN), jnp.float32),
        mesh=mesh,
        scratch_types=[
            pltpu.VMEM((_ROWS_MAX, _NODE_IN), jnp.float32),
            pltpu.VMEM((_ROWS_MAX,), jnp.int32),
            pltpu.VMEM((_B, _NODE_IN), jnp.float32),
            pltpu.SemaphoreType.DMA,
            pltpu.SemaphoreType.DMA,
        ],
    )
    def k(nodes, bat, psum, rows_v, idx_v, acc_v, sem1, sem2):
        cid = lax.axis_index("c")
        sid = lax.axis_index("s")
        w = sid * 2 + cid
        gs = jnp.where(w < _NWBIG, _GBIG * w,
                       _GBIG * _NWBIG + _GSML * (w - _NWBIG))
        big = w < _NWBIG
        rs = gs * _LANES

        h1_rows = _H1 * _LANES
        pltpu.async_copy(nodes.at[pl.ds(rs, h1_rows)],
                         rows_v.at[pl.ds(0, h1_rows)], sem1)

        @pl.when(big)
        def _():
            n2 = (_GBIG - _H1) * _LANES
            pltpu.async_copy(nodes.at[pl.ds(rs + h1_rows, n2)],
                             rows_v.at[pl.ds(h1_rows, n2)], sem2)

        @pl.when(jnp.logical_not(big))
        def _():
            n2 = (_GSML - _H1) * _LANES
            pltpu.async_copy(nodes.at[pl.ds(rs + h1_rows, n2)],
                             rows_v.at[pl.ds(h1_rows, n2)], sem2)

        @pl.when(big)
        def _():
            pltpu.sync_copy(bat.at[pl.ds(rs, _GBIG * _LANES)],
                            idx_v.at[pl.ds(0, _GBIG * _LANES)])

        @pl.when(jnp.logical_not(big))
        def _():
            pltpu.sync_copy(bat.at[pl.ds(rs, _GSML * _LANES)],
                            idx_v.at[pl.ds(0, _GSML * _LANES)])

        zero = jnp.zeros((_LANES,), jnp.float32)

        @plsc.parallel_loop(0, _B)
        def _(r):
            for jj in range(_CBLK):
                acc_v[r, pl.ds(jj * _LANES, _LANES)] = zero

        def load_row(r):
            return [rows_v[r, pl.ds(jj * _LANES, _LANES)]
                    for jj in range(_CBLK)]

        def make_group_body(goff):
            def group_body(g):
                gg = g + goff
                rbase = gg * _LANES
                idvec = idx_v[pl.ds(rbase, _LANES)]
                xs = load_row(rbase)
                for i in range(_LANES):
                    b = idvec[i]
                    nxt = load_row(rbase + i + 1) if i < _LANES - 1 else None
                    for jj in range(_CBLK):
                        plsc.addupdate(
                            acc_v.at[b, pl.ds(jj * _LANES, _LANES)], xs[jj])
                    xs = nxt
            return group_body

        pltpu.make_async_copy(nodes.at[pl.ds(rs, h1_rows)],
                              rows_v.at[pl.ds(0, h1_rows)], sem1).wait()
        plsc.parallel_loop(0, _H1)(make_group_body(0))

        @pl.when(big)
        def _():
            n2 = (_GBIG - _H1) * _LANES
            pltpu.make_async_copy(nodes.at[pl.ds(rs + h1_rows, n2)],
                                  rows_v.at[pl.ds(h1_rows, n2)], sem2).wait()
            plsc.parallel_loop(0, _GBIG - _H1)(make_group_body(_H1))

        @pl.when(jnp.logical_not(big))
        def _():
            n2 = (_GSML - _H1) * _LANES
            pltpu.make_async_copy(nodes.at[pl.ds(rs + h1_rows, n2)],
                                  rows_v.at[pl.ds(h1_rows, n2)], sem2).wait()
            plsc.parallel_loop(0, _GSML - _H1)(make_group_body(_H1))

        pltpu.sync_copy(acc_v, psum.at[w])

    return k(node_feats, batch)


def _tc_mlp_body(ps, bp, g, w0, bb0, w1, bb1, w2, bb2, out):
    sums = jnp.sum(ps[...], axis=0)
    ids = bp[...][:, :, None]
    seg = lax.broadcasted_iota(jnp.int32, (_BROWS, _B, _B), 2)
    cnt = jnp.sum((ids == seg).astype(jnp.float32), axis=(0, 1))
    cnt = cnt.reshape(_B, 1)
    mean = sums / jnp.maximum(cnt, 1.0)
    x = jnp.concatenate([mean, g[...]], axis=1)
    h = jnp.dot(x, w0[...], preferred_element_type=jnp.float32)
    h = jnp.maximum(h + bb0[...], 0.0)
    h = jnp.dot(h, w1[...], preferred_element_type=jnp.float32)
    h = jnp.maximum(h + bb1[...], 0.0)
    h = jnp.dot(h, w2[...], preferred_element_type=jnp.float32)
    out[...] = h + bb2[...]


def _tc_mlp(psum, bpad, glob, w0, b0, w1, b1, w2, b2):
    return pl.pallas_call(
        _tc_mlp_body,
        out_shape=jax.ShapeDtypeStruct((_B, _WIDTH), jnp.float32),
    )(psum, bpad, glob, w0, b0, w1, b1, w2, b2)


def kernel(node_feats, edge_index, edge_feats, glob_feats, batch,
           W0, b0, W1, b1, W2, b2):
    del edge_index, edge_feats
    psum = _sc_segment_sums(node_feats, batch)
    bpad = jnp.concatenate(
        [batch, jnp.full((_BROWS * _B - _C,), _BPAD, jnp.int32)]
    ).reshape(_BROWS, _B)
    return _tc_mlp(psum, bpad, glob_feats, W0, b0, W1, b1, W2, b2)


# speedup vs baseline: 1.2329x; 1.0140x over previous
import functools

import jax
import jax.numpy as jnp
from jax import lax
from jax.experimental import pallas as pl
from jax.experimental.pallas import tpu as pltpu
from jax.experimental.pallas import tpu_sc as plsc

_C = 10000
_B = 128
_NODE_IN = 256
_GLOB_IN = 32
_WIDTH = 512

_LANES = 16
_CBLK = _NODE_IN // _LANES
_NG = _C // _LANES
_NW = 32
_GBIG = -(-_NG // _NW)
_GSML = _NG // _NW
_NWBIG = _NG - _GSML * _NW
_H1 = 6
_ROWS_MAX = _GBIG * _LANES
_BPAD = 1 << 20
_BROWS = -(-_C // _B)


def _sc_segment_sums(node_feats, batch):
    mesh = plsc.VectorSubcoreMesh(core_axis_name="c", subcore_axis_name="s")

    @functools.partial(
        pl.kernel,
        out_type=jax.ShapeDtypeStruct((_NW, _B, _NODE_IN), jnp.float32),
        mesh=mesh,
        scratch_types=[
            pltpu.VMEM((_ROWS_MAX, _NODE_IN), jnp.float32),
            pltpu.VMEM((_ROWS_MAX,), jnp.int32),
            pltpu.VMEM((_B, _NODE_IN), jnp.float32),
            pltpu.SemaphoreType.DMA,
            pltpu.SemaphoreType.DMA,
        ],
    )
    def k(nodes, bat, psum, rows_v, idx_v, acc_v, sem1, sem2):
        cid = lax.axis_index("c")
        sid = lax.axis_index("s")
        w = sid * 2 + cid
        gs = jnp.where(w < _NWBIG, _GBIG * w,
                       _GBIG * _NWBIG + _GSML * (w - _NWBIG))
        big = w < _NWBIG
        rs = gs * _LANES

        h1_rows = _H1 * _LANES
        pltpu.async_copy(nodes.at[pl.ds(rs, h1_rows)],
                         rows_v.at[pl.ds(0, h1_rows)], sem1)

        @pl.when(big)
        def _():
            n2 = (_GBIG - _H1) * _LANES
            pltpu.async_copy(nodes.at[pl.ds(rs + h1_rows, n2)],
                             rows_v.at[pl.ds(h1_rows, n2)], sem2)

        @pl.when(jnp.logical_not(big))
        def _():
            n2 = (_GSML - _H1) * _LANES
            pltpu.async_copy(nodes.at[pl.ds(rs + h1_rows, n2)],
                             rows_v.at[pl.ds(h1_rows, n2)], sem2)

        @pl.when(big)
        def _():
            pltpu.sync_copy(bat.at[pl.ds(rs, _GBIG * _LANES)],
                            idx_v.at[pl.ds(0, _GBIG * _LANES)])

        @pl.when(jnp.logical_not(big))
        def _():
            pltpu.sync_copy(bat.at[pl.ds(rs, _GSML * _LANES)],
                            idx_v.at[pl.ds(0, _GSML * _LANES)])

        zero = jnp.zeros((_LANES,), jnp.float32)

        @plsc.parallel_loop(0, _B)
        def _(r):
            for jj in range(_CBLK):
                acc_v[r, pl.ds(jj * _LANES, _LANES)] = zero

        def load_row(r):
            return [rows_v[r, pl.ds(jj * _LANES, _LANES)]
                    for jj in range(_CBLK)]

        def make_group_body(goff):
            def group_body(g):
                gg = g + goff
                rbase = gg * _LANES
                idvec = idx_v[pl.ds(rbase, _LANES)]
                xs = load_row(rbase)
                for i in range(_LANES):
                    b = idvec[i]
                    nxt = load_row(rbase + i + 1) if i < _LANES - 1 else None
                    for jj in range(_CBLK):
                        plsc.addupdate(
                            acc_v.at[b, pl.ds(jj * _LANES, _LANES)], xs[jj])
                    xs = nxt
            return group_body

        pltpu.make_async_copy(nodes.at[pl.ds(rs, h1_rows)],
                              rows_v.at[pl.ds(0, h1_rows)], sem1).wait()
        plsc.parallel_loop(0, _H1)(make_group_body(0))

        @pl.when(big)
        def _():
            n2 = (_GBIG - _H1) * _LANES
            pltpu.make_async_copy(nodes.at[pl.ds(rs + h1_rows, n2)],
                                  rows_v.at[pl.ds(h1_rows, n2)], sem2).wait()
            plsc.parallel_loop(0, _GBIG - _H1)(make_group_body(_H1))

        @pl.when(jnp.logical_not(big))
        def _():
            n2 = (_GSML - _H1) * _LANES
            pltpu.make_async_copy(nodes.at[pl.ds(rs + h1_rows, n2)],
                                  rows_v.at[pl.ds(h1_rows, n2)], sem2).wait()
            plsc.parallel_loop(0, _GSML - _H1)(make_group_body(_H1))

        pltpu.sync_copy(acc_v, psum.at[w])

    return k(node_feats, batch)


def _tc_mlp_body(ps, bp, g, w0, bb0, w1, bb1, w2, bb2, out):
    sums = jnp.sum(ps[...], axis=0)
    ids = bp[...][:, :, None]
    seg = lax.broadcasted_iota(jnp.int32, (_BROWS, _B, _B), 2)
    cnt = jnp.sum((ids == seg).astype(jnp.float32), axis=(0, 1))
    cnt = cnt.reshape(_B, 1)
    mean = sums / jnp.maximum(cnt, 1.0)
    x = jnp.concatenate([mean, g[...]], axis=1)
    h = jnp.dot(x, w0[...], preferred_element_type=jnp.float32)
    h = jnp.maximum(h + bb0[...], 0.0)
    h = jnp.dot(h, w1[...], preferred_element_type=jnp.float32)
    h = jnp.maximum(h + bb1[...], 0.0)
    h = jnp.dot(h, w2[...], preferred_element_type=jnp.float32)
    out[...] = h + bb2[...]


def _tc_mlp(psum, bpad, glob, w0, b0, w1, b1, w2, b2):
    return pl.pallas_call(
        _tc_mlp_body,
        out_shape=jax.ShapeDtypeStruct((_B, _WIDTH), jnp.float32),
    )(psum, bpad, glob, w0, b0, w1, b1, w2, b2)


def kernel(node_feats, edge_index, edge_feats, glob_feats, batch,
           W0, b0, W1, b1, W2, b2):
    del edge_index, edge_feats
    psum = _sc_segment_sums(node_feats, batch)
    bpad = jnp.concatenate(
        [batch, jnp.full((_BROWS * _B - _C,), _BPAD, jnp.int32)]
    ).reshape(_BROWS, _B)
    return _tc_mlp(psum, bpad, glob_feats, W0, b0, W1, b1, W2, b2)


# speedup vs baseline: 1.2483x; 1.0125x over previous
import functools

import jax
import jax.numpy as jnp
from jax import lax
from jax.experimental import pallas as pl
from jax.experimental.pallas import tpu as pltpu
from jax.experimental.pallas import tpu_sc as plsc

_C = 10000
_B = 128
_NODE_IN = 256
_GLOB_IN = 32
_WIDTH = 512

_LANES = 16
_CBLK = _NODE_IN // _LANES
_NG = _C // _LANES
_NW = 32
_GBIG = -(-_NG // _NW)
_GSML = _NG // _NW
_NWBIG = _NG - _GSML * _NW
_H1 = 4
_ROWS_MAX = _GBIG * _LANES
_BPAD = 1 << 20
_BROWS = -(-_C // _B)


def _sc_segment_sums(node_feats, batch):
    mesh = plsc.VectorSubcoreMesh(core_axis_name="c", subcore_axis_name="s")

    @functools.partial(
        pl.kernel,
        out_type=jax.ShapeDtypeStruct((_NW, _B, _NODE_IN), jnp.float32),
        mesh=mesh,
        scratch_types=[
            pltpu.VMEM((_ROWS_MAX, _NODE_IN), jnp.float32),
            pltpu.VMEM((_ROWS_MAX,), jnp.int32),
            pltpu.VMEM((_B, _NODE_IN), jnp.float32),
            pltpu.SemaphoreType.DMA,
            pltpu.SemaphoreType.DMA,
        ],
    )
    def k(nodes, bat, psum, rows_v, idx_v, acc_v, sem1, sem2):
        cid = lax.axis_index("c")
        sid = lax.axis_index("s")
        w = sid * 2 + cid
        gs = jnp.where(w < _NWBIG, _GBIG * w,
                       _GBIG * _NWBIG + _GSML * (w - _NWBIG))
        big = w < _NWBIG
        rs = gs * _LANES

        h1_rows = _H1 * _LANES
        pltpu.async_copy(nodes.at[pl.ds(rs, h1_rows)],
                         rows_v.at[pl.ds(0, h1_rows)], sem1)

        @pl.when(big)
        def _():
            n2 = (_GBIG - _H1) * _LANES
            pltpu.async_copy(nodes.at[pl.ds(rs + h1_rows, n2)],
                             rows_v.at[pl.ds(h1_rows, n2)], sem2)

        @pl.when(jnp.logical_not(big))
        def _():
            n2 = (_GSML - _H1) * _LANES
            pltpu.async_copy(nodes.at[pl.ds(rs + h1_rows, n2)],
                             rows_v.at[pl.ds(h1_rows, n2)], sem2)

        @pl.when(big)
        def _():
            pltpu.sync_copy(bat.at[pl.ds(rs, _GBIG * _LANES)],
                            idx_v.at[pl.ds(0, _GBIG * _LANES)])

        @pl.when(jnp.logical_not(big))
        def _():
            pltpu.sync_copy(bat.at[pl.ds(rs, _GSML * _LANES)],
                            idx_v.at[pl.ds(0, _GSML * _LANES)])

        zero = jnp.zeros((_LANES,), jnp.float32)

        @plsc.parallel_loop(0, _B)
        def _(r):
            for jj in range(_CBLK):
                acc_v[r, pl.ds(jj * _LANES, _LANES)] = zero

        def load_row(r):
            return [rows_v[r, pl.ds(jj * _LANES, _LANES)]
                    for jj in range(_CBLK)]

        def make_group_body(goff):
            def group_body(g):
                gg = g + goff
                rbase = gg * _LANES
                idvec = idx_v[pl.ds(rbase, _LANES)]
                xs = load_row(rbase)
                for i in range(_LANES):
                    b = idvec[i]
                    nxt = load_row(rbase + i + 1) if i < _LANES - 1 else None
                    for jj in range(_CBLK):
                        plsc.addupdate(
                            acc_v.at[b, pl.ds(jj * _LANES, _LANES)], xs[jj])
                    xs = nxt
            return group_body

        pltpu.make_async_copy(nodes.at[pl.ds(rs, h1_rows)],
                              rows_v.at[pl.ds(0, h1_rows)], sem1).wait()
        plsc.parallel_loop(0, _H1)(make_group_body(0))

        @pl.when(big)
        def _():
            n2 = (_GBIG - _H1) * _LANES
            pltpu.make_async_copy(nodes.at[pl.ds(rs + h1_rows, n2)],
                                  rows_v.at[pl.ds(h1_rows, n2)], sem2).wait()
            plsc.parallel_loop(0, _GBIG - _H1)(make_group_body(_H1))

        @pl.when(jnp.logical_not(big))
        def _():
            n2 = (_GSML - _H1) * _LANES
            pltpu.make_async_copy(nodes.at[pl.ds(rs + h1_rows, n2)],
                                  rows_v.at[pl.ds(h1_rows, n2)], sem2).wait()
            plsc.parallel_loop(0, _GSML - _H1)(make_group_body(_H1))

        pltpu.sync_copy(acc_v, psum.at[w])

    return k(node_feats, batch)


def _tc_mlp_body(ps, bp, g, w0, bb0, w1, bb1, w2, bb2, out):
    sums = jnp.sum(ps[...], axis=0)
    ids = bp[...][:, :, None]
    seg = lax.broadcasted_iota(jnp.int32, (_BROWS, _B, _B), 2)
    cnt = jnp.sum((ids == seg).astype(jnp.float32), axis=(0, 1))
    cnt = cnt.reshape(_B, 1)
    mean = sums / jnp.maximum(cnt, 1.0)
    x = jnp.concatenate([mean, g[...]], axis=1)
    h = jnp.dot(x, w0[...], preferred_element_type=jnp.float32)
    h = jnp.maximum(h + bb0[...], 0.0)
    h = jnp.dot(h, w1[...], preferred_element_type=jnp.float32)
    h = jnp.maximum(h + bb1[...], 0.0)
    h = jnp.dot(h, w2[...], preferred_element_type=jnp.float32)
    out[...] = h + bb2[...]


def _tc_mlp(psum, bpad, glob, w0, b0, w1, b1, w2, b2):
    return pl.pallas_call(
        _tc_mlp_body,
        out_shape=jax.ShapeDtypeStruct((_B, _WIDTH), jnp.float32),
    )(psum, bpad, glob, w0, b0, w1, b1, w2, b2)


def kernel(node_feats, edge_index, edge_feats, glob_feats, batch,
           W0, b0, W1, b1, W2, b2):
    del edge_index, edge_feats
    psum = _sc_segment_sums(node_feats, batch)
    bpad = jnp.concatenate(
        [batch, jnp.full((_BROWS * _B - _C,), _BPAD, jnp.int32)]
    ).reshape(_BROWS, _B)
    return _tc_mlp(psum, bpad, glob_feats, W0, b0, W1, b1, W2, b2)


# speedup vs baseline: 1.2636x; 1.0123x over previous
import functools

import jax
import jax.numpy as jnp
from jax import lax
from jax.experimental import pallas as pl
from jax.experimental.pallas import tpu as pltpu
from jax.experimental.pallas import tpu_sc as plsc

_C = 10000
_B = 128
_NODE_IN = 256
_GLOB_IN = 32
_WIDTH = 512

_LANES = 16
_CBLK = _NODE_IN // _LANES
_NG = _C // _LANES
_NW = 32
_GBIG = -(-_NG // _NW)
_GSML = _NG // _NW
_NWBIG = _NG - _GSML * _NW
_H1 = 2
_ROWS_MAX = _GBIG * _LANES
_BPAD = 1 << 20
_BROWS = -(-_C // _B)


def _sc_segment_sums(node_feats, batch):
    mesh = plsc.VectorSubcoreMesh(core_axis_name="c", subcore_axis_name="s")

    @functools.partial(
        pl.kernel,
        out_type=jax.ShapeDtypeStruct((_NW, _B, _NODE_IN), jnp.float32),
        mesh=mesh,
        scratch_types=[
            pltpu.VMEM((_ROWS_MAX, _NODE_IN), jnp.float32),
            pltpu.VMEM((_ROWS_MAX,), jnp.int32),
            pltpu.VMEM((_B, _NODE_IN), jnp.float32),
            pltpu.SemaphoreType.DMA,
            pltpu.SemaphoreType.DMA,
        ],
    )
    def k(nodes, bat, psum, rows_v, idx_v, acc_v, sem1, sem2):
        cid = lax.axis_index("c")
        sid = lax.axis_index("s")
        w = sid * 2 + cid
        gs = jnp.where(w < _NWBIG, _GBIG * w,
                       _GBIG * _NWBIG + _GSML * (w - _NWBIG))
        big = w < _NWBIG
        rs = gs * _LANES

        h1_rows = _H1 * _LANES
        pltpu.async_copy(nodes.at[pl.ds(rs, h1_rows)],
                         rows_v.at[pl.ds(0, h1_rows)], sem1)

        @pl.when(big)
        def _():
            n2 = (_GBIG - _H1) * _LANES
            pltpu.async_copy(nodes.at[pl.ds(rs + h1_rows, n2)],
                             rows_v.at[pl.ds(h1_rows, n2)], sem2)

        @pl.when(jnp.logical_not(big))
        def _():
            n2 = (_GSML - _H1) * _LANES
            pltpu.async_copy(nodes.at[pl.ds(rs + h1_rows, n2)],
                             rows_v.at[pl.ds(h1_rows, n2)], sem2)

        @pl.when(big)
        def _():
            pltpu.sync_copy(bat.at[pl.ds(rs, _GBIG * _LANES)],
                            idx_v.at[pl.ds(0, _GBIG * _LANES)])

        @pl.when(jnp.logical_not(big))
        def _():
            pltpu.sync_copy(bat.at[pl.ds(rs, _GSML * _LANES)],
                            idx_v.at[pl.ds(0, _GSML * _LANES)])

        zero = jnp.zeros((_LANES,), jnp.float32)

        @plsc.parallel_loop(0, _B)
        def _(r):
            for jj in range(_CBLK):
                acc_v[r, pl.ds(jj * _LANES, _LANES)] = zero

        def load_row(r):
            return [rows_v[r, pl.ds(jj * _LANES, _LANES)]
                    for jj in range(_CBLK)]

        def make_group_body(goff):
            def group_body(g):
                gg = g + goff
                rbase = gg * _LANES
                idvec = idx_v[pl.ds(rbase, _LANES)]
                xs = load_row(rbase)
                for i in range(_LANES):
                    b = idvec[i]
                    nxt = load_row(rbase + i + 1) if i < _LANES - 1 else None
                    for jj in range(_CBLK):
                        plsc.addupdate(
                            acc_v.at[b, pl.ds(jj * _LANES, _LANES)], xs[jj])
                    xs = nxt
            return group_body

        pltpu.make_async_copy(nodes.at[pl.ds(rs, h1_rows)],
                              rows_v.at[pl.ds(0, h1_rows)], sem1).wait()
        plsc.parallel_loop(0, _H1)(make_group_body(0))

        @pl.when(big)
        def _():
            n2 = (_GBIG - _H1) * _LANES
            pltpu.make_async_copy(nodes.at[pl.ds(rs + h1_rows, n2)],
                                  rows_v.at[pl.ds(h1_rows, n2)], sem2).wait()
            plsc.parallel_loop(0, _GBIG - _H1)(make_group_body(_H1))

        @pl.when(jnp.logical_not(big))
        def _():
            n2 = (_GSML - _H1) * _LANES
            pltpu.make_async_copy(nodes.at[pl.ds(rs + h1_rows, n2)],
                                  rows_v.at[pl.ds(h1_rows, n2)], sem2).wait()
            plsc.parallel_loop(0, _GSML - _H1)(make_group_body(_H1))

        pltpu.sync_copy(acc_v, psum.at[w])

    return k(node_feats, batch)


def _tc_mlp_body(ps, bp, g, w0, bb0, w1, bb1, w2, bb2, out):
    sums = jnp.sum(ps[...], axis=0)
    ids = bp[...][:, :, None]
    seg = lax.broadcasted_iota(jnp.int32, (_BROWS, _B, _B), 2)
    cnt = jnp.sum((ids == seg).astype(jnp.float32), axis=(0, 1))
    cnt = cnt.reshape(_B, 1)
    mean = sums / jnp.maximum(cnt, 1.0)
    x = jnp.concatenate([mean, g[...]], axis=1)
    h = jnp.dot(x, w0[...], preferred_element_type=jnp.float32)
    h = jnp.maximum(h + bb0[...], 0.0)
    h = jnp.dot(h, w1[...], preferred_element_type=jnp.float32)
    h = jnp.maximum(h + bb1[...], 0.0)
    h = jnp.dot(h, w2[...], preferred_element_type=jnp.float32)
    out[...] = h + bb2[...]


def _tc_mlp(psum, bpad, glob, w0, b0, w1, b1, w2, b2):
    return pl.pallas_call(
        _tc_mlp_body,
        out_shape=jax.ShapeDtypeStruct((_B, _WIDTH), jnp.float32),
    )(psum, bpad, glob, w0, b0, w1, b1, w2, b2)


def kernel(node_feats, edge_index, edge_feats, glob_feats, batch,
           W0, b0, W1, b1, W2, b2):
    del edge_index, edge_feats
    psum = _sc_segment_sums(node_feats, batch)
    bpad = jnp.concatenate(
        [batch, jnp.full((_BROWS * _B - _C,), _BPAD, jnp.int32)]
    ).reshape(_BROWS, _B)
    return _tc_mlp(psum, bpad, glob_feats, W0, b0, W1, b1, W2, b2)
